# baseline (speedup 1.0000x reference)
import jax
import jax.numpy as jnp
from jax.experimental import pallas as pl
from jax.experimental.pallas import tpu as pltpu

_S1 = 104
_P = 4


def _v2v_kernel(x_ref, w1_ref, b1_ref, w2_ref, b2_ref, w3_ref, b3_ref,
                wf1_ref, bf1_ref, wf2_ref, bf2_ref, out_ref,
                a1_ref, p1_ref, a2a_ref, a2b_ref, p2_ref,
                a30_ref, a31_ref, a32_ref, a33_ref):
    f32 = jnp.float32
    bf16 = jnp.bfloat16
    tn = out_ref.shape[0]
    th = tn // _P
    a2_refs = [a2a_ref, a2b_ref]
    a3_refs = [a30_ref, a31_ref, a32_ref, a33_ref]

    def conv_lrelu(in_ref, w_ref, b_ref, out_scrs, rows):
        K = w_ref.shape[0]
        cout = w_ref.shape[2]
        ch_max = max(8, 24576 // cout)
        r0 = 0
        while r0 < rows:
            ch = min(ch_max, rows - r0)
            acc = jnp.dot(in_ref[pl.ds(r0, ch), :], w_ref[0],
                          preferred_element_type=f32)
            for k in range(1, K):
                acc = acc + jnp.dot(in_ref[pl.ds(r0 + k, ch), :], w_ref[k],
                                    preferred_element_type=f32)
            z = acc + b_ref[...]
            z = jnp.maximum(z, 0.01 * z)
            g = cout // len(out_scrs)
            for i, scr in enumerate(out_scrs):
                scr[pl.ds(r0, ch), :] = z[:, i * g:(i + 1) * g]
            r0 += ch

    def maxpool2(in_scrs, out_scr):
        m = out_scr.shape[0] - 8
        g = out_scr.shape[1] // len(in_scrs)
        for i, scr in enumerate(in_scrs):
            mx = jnp.maximum(scr[pl.ds(0, m, 2), :], scr[pl.ds(1, m, 2), :])
            out_scr[pl.ds(0, m), pl.ds(i * g, g)] = mx.astype(bf16)
        out_scr[pl.ds(m, 8), :] = jnp.zeros((8, out_scr.shape[1]), bf16)

    r1 = a1_ref.shape[0]
    conv_lrelu(x_ref, w1_ref, b1_ref, [a1_ref], r1 - 8)
    a1_ref[pl.ds(r1 - 8, 8), :] = jnp.zeros((8, a1_ref.shape[1]), f32)

    maxpool2([a1_ref], p1_ref)
    conv_lrelu(p1_ref, w2_ref, b2_ref, a2_refs, a2a_ref.shape[0])
    maxpool2(a2_refs, p2_ref)
    conv_lrelu(p2_ref, w3_ref, b3_ref, a3_refs, a30_ref.shape[0])

    slot3 = a30_ref.shape[0] // th
    hidden = wf1_ref.shape[2]
    cells = []
    for j in range(_P):
        aj = a3_refs[j]
        acc = jnp.zeros((th, hidden), f32)
        for l in range(wf1_ref.shape[0]):
            rows = jnp.maximum(aj[pl.ds(2 * l, th, slot3), :],
                               aj[pl.ds(2 * l + 1, th, slot3), :])
            acc = acc + jnp.dot(rows.astype(bf16), wf1_ref[l],
                                preferred_element_type=f32)
        cells.append(acc)
    f1 = jnp.maximum(jnp.concatenate(cells, axis=0) + bf1_ref[...], 0.0)

    f2 = jnp.tanh(jnp.dot(f1.astype(bf16), wf2_ref[...],
                          preferred_element_type=f32) + bf2_ref[...])
    inv = jax.lax.rsqrt(jnp.sum(f2 * f2, axis=-1, keepdims=True) + 1e-12)
    out_ref[...] = f2 * inv


def _blockdiag(w, p):
    eye = jnp.eye(p, dtype=w.dtype)
    return jax.vmap(lambda wk: jnp.kron(eye, wk))(w).astype(jnp.bfloat16)


def kernel(w1, b1, w2, b2, w3, b3, wf1, bf1, wf2, bf2, x):
    B, three, F_, T = x.shape
    N = B * three
    dim = wf2.shape[1]
    tile_n = 64
    tn = max(_P * 8, (min(tile_n, N) + _P * 8 - 1) // (_P * 8) * (_P * 8))
    th = tn // _P
    n_pad = (N + tn - 1) // tn * tn
    n_tiles = n_pad // tn

    xt = jnp.transpose(x.reshape(N, F_, T), (0, 2, 1))
    xt = jnp.pad(xt, ((0, n_pad - N), (0, _S1 - T), (0, 0)))
    xt = xt.reshape(n_tiles, _P, th, _S1, F_).transpose(0, 2, 3, 1, 4)
    x2d = xt.reshape(n_tiles * th * _S1, _P * F_).astype(jnp.bfloat16)

    w1b = _blockdiag(w1, _P)
    w2b = _blockdiag(w2, _P)
    w3b = _blockdiag(w3, _P)
    b1b = jnp.tile(b1, (1, _P))
    b2b = jnp.tile(b2, (1, _P))
    b3b = jnp.tile(b3, (1, _P))
    wf1b = wf1.astype(jnp.bfloat16)
    wf2b = wf2.astype(jnp.bfloat16)

    full = lambda a: pl.BlockSpec(a.shape, lambda i, _nd=a.ndim: (0,) * _nd)

    out = pl.pallas_call(
        _v2v_kernel,
        out_shape=jax.ShapeDtypeStruct((n_pad, dim), jnp.float32),
        grid=(n_tiles,),
        in_specs=[
            pl.BlockSpec((th * _S1, _P * F_), lambda i: (i, 0)),
            full(w1b), full(b1b),
            full(w2b), full(b2b),
            full(w3b), full(b3b),
            full(wf1b), full(bf1),
            full(wf2b), full(bf2),
        ],
        out_specs=pl.BlockSpec((tn, dim), lambda i: (i, 0)),
        scratch_shapes=[
            pltpu.VMEM((th * _S1, _P * 32), jnp.float32),
            pltpu.VMEM((th * 52 + 8, _P * 32), jnp.bfloat16),
            pltpu.VMEM((th * 52, 128), jnp.float32),
            pltpu.VMEM((th * 52, 128), jnp.float32),
            pltpu.VMEM((th * 26 + 8, _P * 64), jnp.bfloat16),
            pltpu.VMEM((th * 26, 128), jnp.float32),
            pltpu.VMEM((th * 26, 128), jnp.float32),
            pltpu.VMEM((th * 26, 128), jnp.float32),
            pltpu.VMEM((th * 26, 128), jnp.float32),
        ],
        compiler_params=pltpu.CompilerParams(
            dimension_semantics=("parallel",),
            vmem_limit_bytes=64 * 1024 * 1024),
    )(x2d, w1b, b1b, w2b, b2b, w3b, b3b, wf1b, bf1, wf2b, bf2)

    return out[:N].reshape(B, three, dim)
